# baseline (speedup 1.0000x reference)
import jax
import jax.numpy as jnp
from jax.experimental import pallas as pl

_B_BLOCK = 16384


def _bcast_kernel(w_ref, o_ref):
    l = pl.program_id(0)
    row = w_ref[pl.ds(l, 1), :]
    o_ref[...] = jnp.broadcast_to(row[:, None, :], o_ref.shape)


def kernel(x, pos_embed_weight):
    B = x.shape[0]
    L, C = pos_embed_weight.shape
    lbc = pl.pallas_call(
        _bcast_kernel,
        grid=(L, B // _B_BLOCK),
        in_specs=[pl.BlockSpec((L, C), lambda l, i: (0, 0))],
        out_specs=pl.BlockSpec((1, _B_BLOCK, C), lambda l, i: (l, i, 0)),
        out_shape=jax.ShapeDtypeStruct((L, B, C), jnp.float32),
    )(pos_embed_weight)
    return jnp.transpose(lbc, (1, 2, 0))


# speedup vs baseline: 1.0130x; 1.0130x over previous
import jax
import jax.numpy as jnp
from jax.experimental import pallas as pl

_B_BLOCK = 4096


def _bcast_kernel(w_ref, o_ref):
    l = pl.program_id(0)
    row = w_ref[pl.ds(l, 1), :]
    o_ref[...] = jnp.broadcast_to(row[:, None, :], o_ref.shape)


def kernel(x, pos_embed_weight):
    B = x.shape[0]
    L, C = pos_embed_weight.shape
    lbc = pl.pallas_call(
        _bcast_kernel,
        grid=(L, B // _B_BLOCK),
        in_specs=[pl.BlockSpec((L, C), lambda l, i: (0, 0))],
        out_specs=pl.BlockSpec((1, _B_BLOCK, C), lambda l, i: (l, i, 0)),
        out_shape=jax.ShapeDtypeStruct((L, B, C), jnp.float32),
    )(pos_embed_weight)
    return jnp.transpose(lbc, (1, 2, 0))
